# Initial kernel scaffold; baseline (speedup 1.0000x reference)
import jax
import jax.numpy as jnp
from jax.experimental import pallas as pl


def kernel(t):
    raise NotImplementedError("write your pallas kernel here")


# speedup vs baseline: 1.3204x; 1.3204x over previous
import jax
import jax.numpy as jnp
from jax.experimental import pallas as pl

_N_CLASSES = 1000
_BB = 8


def _onehot_block(t_ref, out_ref):
    t = t_ref[...]
    c = jax.lax.broadcasted_iota(jnp.int32, out_ref.shape, 1)
    out_ref[...] = (c == t[:, None, :]).astype(jnp.float32)


def kernel(t) -> jnp.ndarray:
    B, S = t.shape
    C = _N_CLASSES
    bb = _BB
    grid = (B // bb,)
    return pl.pallas_call(
        _onehot_block,
        grid=grid,
        in_specs=[pl.BlockSpec((bb, S), lambda i: (i, 0))],
        out_specs=pl.BlockSpec((bb, C, S), lambda i: (i, 0, 0)),
        out_shape=jax.ShapeDtypeStruct((B, C, S), jnp.float32),
    )(t.astype(jnp.int32))
